# baseline (speedup 1.0000x reference)
import functools

import jax
import jax.numpy as jnp
from jax import lax
from jax.experimental import pallas as pl
from jax.experimental.pallas import tpu as pltpu
from jax.experimental.pallas import tpu_sc as plsc

N = 10000
E = 320000
D = 128

NP = 10240
ROWS_PER_TILE = NP // 16
NSUB = 32
CHUNK = 128
CHUNKS_PER_SUB = 80
EDGES_PER_SUB = CHUNKS_PER_SUB * CHUNK
EP = NSUB * EDGES_PER_SUB
NITER = CHUNKS_PER_SUB // 2

_f32 = jnp.float32
_i32 = jnp.int32

_SC_PARAMS = pltpu.CompilerParams(use_tc_tiling_on_sc=False,
                                  needs_layout_passes=False)
_SC_MESH = plsc.VectorSubcoreMesh(core_axis_name="c", subcore_axis_name="s")


def _tc1_body(x_ref, w1_ref, as_m_ref, ad_m_ref, t1h_ref, t1d_ref, c_ref):
    h = jnp.dot(x_ref[...], w1_ref[...], preferred_element_type=_f32)
    a_s = jnp.dot(h, as_m_ref[...], preferred_element_type=_f32)
    a_d = jnp.dot(h, ad_m_ref[...], preferred_element_type=_f32)
    c = jnp.max(a_s) + jnp.max(a_d)
    t1h_ref[...] = jnp.zeros((NP, 80), _f32)
    t1h_ref[0:N, 0:64] = h
    t1h_ref[0:N, 64:72] = a_s
    t1d_ref[...] = jnp.zeros((NP, 16), _f32)
    t1d_ref[0:N, 0:8] = a_d
    c_ref[...] = jnp.full((1, 128), c, _f32)


def _tc2_body(s1p_ref, t1h_ref, t1d_ref, c1_ref, b1_ref, w2_ref, as2_ref,
              ad2_ref, exp8_ref, t2h_ref, t2d_ref, c2_ref):
    s = s1p_ref[0] + s1p_ref[1]
    a_s = t1h_ref[0:N, 64:72]
    a_d = t1d_ref[0:N, 0:8]
    c1 = c1_ref[0, 0]
    z = a_s + a_d
    ex_self = jnp.exp(jnp.maximum(z, 0.2 * z) - c1)
    denom = s[0:N, 64:72] + ex_self
    h1 = t1h_ref[0:N, 0:64]
    num = s[0:N, 0:64] + jnp.dot(ex_self, exp8_ref[...],
                                 preferred_element_type=_f32) * h1
    dexp = jnp.dot(denom, exp8_ref[...], preferred_element_type=_f32)
    out1 = num / dexp + b1_ref[...]
    h2 = jnp.dot(out1, w2_ref[...], preferred_element_type=_f32)
    a_s2 = jnp.dot(h2, as2_ref[...], preferred_element_type=_f32)
    a_d2 = jnp.dot(h2, ad2_ref[...], preferred_element_type=_f32)
    c2 = jnp.max(a_s2) + jnp.max(a_d2)
    t2h_ref[...] = jnp.zeros((NP, 16), _f32)
    t2h_ref[0:N, 0:7] = h2
    t2h_ref[0:N, 7:8] = a_s2
    t2d_ref[...] = jnp.zeros((NP, 16), _f32)
    t2d_ref[0:N, 0:1] = a_d2
    c2_ref[...] = jnp.full((1, 128), c2, _f32)


def _tc3_body(s2p_ref, t2h_ref, t2d_ref, c2_ref, b2_ref, out_ref):
    s = s2p_ref[0] + s2p_ref[1]
    h2 = t2h_ref[0:N, 0:7]
    a_s2 = t2h_ref[0:N, 7:8]
    a_d2 = t2d_ref[0:N, 0:1]
    c2 = c2_ref[0, 0]
    z = a_s2 + a_d2
    ex_self = jnp.exp(jnp.maximum(z, 0.2 * z) - c2)
    num = s[0:N, 0:7] + ex_self * h2
    den = s[0:N, 7:8] + ex_self
    o = num / den + b2_ref[...]
    m = jnp.max(o, axis=1, keepdims=True)
    zz = o - m
    out_ref[...] = zz - jnp.log(jnp.sum(jnp.exp(zz), axis=1, keepdims=True))


def _make_sc(width, edge_fn):
    @functools.partial(
        pl.kernel,
        mesh=_SC_MESH,
        compiler_params=_SC_PARAMS,
        out_type=jax.ShapeDtypeStruct((2, NP, width), _f32),
        scratch_types=(
            [pltpu.VMEM((CHUNK, width), _f32)] * 4 +
            [pltpu.VMEM((CHUNK, 16), _f32)] * 4 +
            [pltpu.VMEM((CHUNK, width), _f32)] * 2 +
            [pltpu.VMEM((CHUNK,), _i32)] * 8 +
            [pltpu.VMEM((CHUNK,), _i32)] * 8 +
            [pltpu.VMEM((16,), _f32)] * 2 +
            [pltpu.VMEM_SHARED((NP, width), _f32)] +
            [pltpu.SemaphoreType.DMA] * 10
        ),
    )
    def sck(th_hbm, td_hbm, src_hbm, dst_hbm, cvec_hbm, zeros_hbm,
            out_hbm, *scr):
        sbufs = scr[0:4]
        dbufs = scr[4:8]
        obufs = scr[8:10]
        idxss = scr[10:18]
        idxds = scr[18:26]
        exb = scr[26]
        cbuf = scr[27]
        acc = scr[28]
        gsems = scr[29:33]
        ssems = scr[33:35]
        isems = scr[35:39]

        cidx = lax.axis_index("c")
        sidx = lax.axis_index("s")
        wid = cidx * 16 + sidx

        r0 = sidx * ROWS_PER_TILE
        pltpu.sync_copy(zeros_hbm.at[pl.ds(r0, ROWS_PER_TILE)],
                        acc.at[pl.ds(r0, ROWS_PER_TILE)])
        pltpu.sync_copy(cvec_hbm, cbuf)
        plsc.subcore_barrier()

        def start_idx(t, k8, k4):
            pltpu.async_copy(src_hbm.at[wid, t], idxss[k8], isems[k4])
            pltpu.async_copy(dst_hbm.at[wid, t], idxds[k8], isems[k4])

        def start_gathers(t, k8, k4):
            pltpu.make_async_copy(src_hbm.at[wid, t], idxss[k8],
                                  isems[k4]).wait()
            pltpu.make_async_copy(dst_hbm.at[wid, t], idxds[k8],
                                  isems[k4]).wait()
            pltpu.async_copy(th_hbm.at[idxss[k8]], sbufs[k4], gsems[k4])
            pltpu.async_copy(td_hbm.at[idxds[k8]], dbufs[k4], gsems[k4])

        def wait_gathers(k8, k4):
            pltpu.make_async_copy(th_hbm.at[idxss[k8]], sbufs[k4],
                                  gsems[k4]).wait()
            pltpu.make_async_copy(td_hbm.at[idxds[k8]], dbufs[k4],
                                  gsems[k4]).wait()

        def wait_scatter(q):
            pltpu.make_async_copy(obufs[q], acc.at[idxds[q]],
                                  ssems[q]).wait()

        def do_chunk(c, k):
            k4, q = k % 4, k % 2
            wait_gathers(k, k4)

            def edge_body(i, carry):
                for u in range(4):
                    edge_fn(4 * i + u, sbufs[k4], dbufs[k4], obufs[q],
                            cbuf, exb)
                return carry

            lax.fori_loop(0, CHUNK // 4, edge_body, 0)
            pltpu.async_copy(obufs[q], acc.at[idxds[k]], ssems[q], add=True)

            @pl.when(c + 4 < CHUNKS_PER_SUB)
            def _():
                start_idx(c + 4, (k + 4) % 8, k4)

            @pl.when(c + 2 < CHUNKS_PER_SUB)
            def _():
                start_gathers(c + 2, (k + 2) % 8, (k + 2) % 4)

        for t in range(4):
            start_idx(t, t, t)
        start_gathers(0, 0, 0)
        start_gathers(1, 1, 1)

        def iter_body(g, carry):
            c0 = 8 * g
            for k in range(8):
                if k >= 2:
                    wait_scatter(k % 2)
                else:
                    @pl.when(g > 0)
                    def _():
                        wait_scatter(k % 2)
                do_chunk(c0 + k, k)
            return carry

        lax.fori_loop(0, CHUNKS_PER_SUB // 8, iter_body, 0)
        wait_scatter(0)
        wait_scatter(1)
        plsc.subcore_barrier()
        pltpu.sync_copy(acc.at[pl.ds(r0, ROWS_PER_TILE)],
                        out_hbm.at[cidx, pl.ds(r0, ROWS_PER_TILE)])

    return sck


_GDN = lax.GatherDimensionNumbers(offset_dims=(), collapsed_slice_dims=(0,),
                                  start_index_map=(0,))


def _vgather(vec, idx):
    return lax.gather(vec, idx[:, None], _GDN, (1,),
                      mode=lax.GatherScatterMode.PROMISE_IN_BOUNDS)


def _edge1(i, sbuf, dbuf, obuf, cbuf, exb):
    cv = cbuf[...]
    ge8 = lax.iota(_i32, 16) // 8
    sv4 = sbuf[i, pl.ds(64, 16)]
    dv = dbuf[i, pl.ds(0, 16)]
    z = sv4 + dv
    ex = jnp.exp(jnp.maximum(z, 0.2 * z) - cv)
    for k in range(4):
        exk = _vgather(ex, ge8 + 2 * k)
        obuf[i, pl.ds(16 * k, 16)] = sbuf[i, pl.ds(16 * k, 16)] * exk
    obuf[i, pl.ds(64, 16)] = ex


def _edge2(i, sbuf, dbuf, obuf, cbuf, exb):
    cv = cbuf[...]
    iot = lax.iota(_i32, 16)
    idx7 = jnp.full((16,), 7, _i32)
    idx0 = jnp.zeros((16,), _i32)
    mask7 = jnp.clip(7 - iot, 0, 1).astype(_f32)
    one7 = (jnp.clip(8 - iot, 0, 1)
            - jnp.clip(7 - iot, 0, 1)).astype(_f32)
    srow = sbuf[i, pl.ds(0, 16)]
    drow = dbuf[i, pl.ds(0, 16)]
    asv = _vgather(srow, idx7)
    adv = _vgather(drow, idx0)
    z = asv + adv
    ex = jnp.exp(jnp.maximum(z, 0.2 * z) - cv)
    obuf[i, pl.ds(0, 16)] = (srow * mask7 + one7) * ex


_SCK1 = _make_sc(80, _edge1)
_SCK2 = _make_sc(16, _edge2)


def kernel(x, edge_index, W1, att_src1, att_dst1, b1, W2, att_src2, att_dst2,
           b2):
    lane = jnp.arange(64)
    head = lane // 8
    blk = (head[:, None] == jnp.arange(8)[None, :]).astype(_f32)
    as_m = blk * att_src1.reshape(64)[:, None]
    ad_m = blk * att_dst1.reshape(64)[:, None]
    exp8 = blk.T
    as2_m = att_src2.reshape(7, 1)
    ad2_m = att_dst2.reshape(7, 1)

    pad = jnp.full((EP - E,), N, _i32)
    src3d = jnp.concatenate([edge_index[0].astype(_i32), pad]).reshape(
        NSUB, CHUNKS_PER_SUB, CHUNK)
    dst3d = jnp.concatenate([edge_index[1].astype(_i32), pad]).reshape(
        NSUB, CHUNKS_PER_SUB, CHUNK)

    zeros80 = jnp.zeros((NP, 80), _f32)
    zeros16 = jnp.zeros((NP, 16), _f32)

    t1h, t1d, c1 = pl.pallas_call(
        _tc1_body,
        out_shape=[
            jax.ShapeDtypeStruct((NP, 80), _f32),
            jax.ShapeDtypeStruct((NP, 16), _f32),
            jax.ShapeDtypeStruct((1, 128), _f32),
        ],
    )(x, W1, as_m, ad_m)
    c1v = c1[0, :16]

    s1p = _SCK1(t1h, t1d, src3d, dst3d, c1v, zeros80)

    t2h, t2d, c2 = pl.pallas_call(
        _tc2_body,
        out_shape=[
            jax.ShapeDtypeStruct((NP, 16), _f32),
            jax.ShapeDtypeStruct((NP, 16), _f32),
            jax.ShapeDtypeStruct((1, 128), _f32),
        ],
    )(s1p, t1h, t1d, c1, b1, W2, as2_m, ad2_m, exp8)
    c2v = c2[0, :16]

    s2p = _SCK2(t2h, t2d, src3d, dst3d, c2v, zeros16)

    out = pl.pallas_call(
        _tc3_body,
        out_shape=jax.ShapeDtypeStruct((N, 7), _f32),
    )(s2p, t2h, t2d, c2, b2)
    return out


# speedup vs baseline: 1.0032x; 1.0032x over previous
import functools

import jax
import jax.numpy as jnp
from jax import lax
from jax.experimental import pallas as pl
from jax.experimental.pallas import tpu as pltpu
from jax.experimental.pallas import tpu_sc as plsc

N = 10000
E = 320000
D = 128

NP = 10240
ROWS_PER_TILE = NP // 16
NSUB = 32
CHUNK = 128
CHUNKS_PER_SUB = 80
EDGES_PER_SUB = CHUNKS_PER_SUB * CHUNK
EP = NSUB * EDGES_PER_SUB
NITER = CHUNKS_PER_SUB // 2

_f32 = jnp.float32
_i32 = jnp.int32

_SC_PARAMS = pltpu.CompilerParams(use_tc_tiling_on_sc=False,
                                  needs_layout_passes=False)
_SC_MESH = plsc.VectorSubcoreMesh(core_axis_name="c", subcore_axis_name="s")


def _tc1_body(x_ref, w1_ref, as_m_ref, ad_m_ref, t1h_ref, t1d_ref, c_ref):
    h = jnp.dot(x_ref[...], w1_ref[...], preferred_element_type=_f32)
    a_s = jnp.dot(h, as_m_ref[...], preferred_element_type=_f32)
    a_d = jnp.dot(h, ad_m_ref[...], preferred_element_type=_f32)
    c = jnp.max(a_s) + jnp.max(a_d)
    t1h_ref[...] = jnp.zeros((NP, 80), _f32)
    t1h_ref[0:N, 0:64] = h
    t1h_ref[0:N, 64:72] = a_s
    t1d_ref[...] = jnp.zeros((NP, 16), _f32)
    t1d_ref[0:N, 0:8] = a_d
    c_ref[...] = jnp.full((1, 128), c, _f32)


def _tc2_body(s1p_ref, t1h_ref, t1d_ref, c1_ref, b1_ref, w2_ref, as2_ref,
              ad2_ref, exp8_ref, t2h_ref, t2d_ref, c2_ref):
    s = s1p_ref[0] + s1p_ref[1]
    a_s = t1h_ref[0:N, 64:72]
    a_d = t1d_ref[0:N, 0:8]
    c1 = c1_ref[0, 0]
    z = a_s + a_d
    ex_self = jnp.exp(jnp.maximum(z, 0.2 * z) - c1)
    denom = s[0:N, 64:72] + ex_self
    h1 = t1h_ref[0:N, 0:64]
    num = s[0:N, 0:64] + jnp.dot(ex_self, exp8_ref[...],
                                 preferred_element_type=_f32) * h1
    dexp = jnp.dot(denom, exp8_ref[...], preferred_element_type=_f32)
    out1 = num / dexp + b1_ref[...]
    h2 = jnp.dot(out1, w2_ref[...], preferred_element_type=_f32)
    a_s2 = jnp.dot(h2, as2_ref[...], preferred_element_type=_f32)
    a_d2 = jnp.dot(h2, ad2_ref[...], preferred_element_type=_f32)
    c2 = jnp.max(a_s2) + jnp.max(a_d2)
    t2h_ref[...] = jnp.zeros((NP, 16), _f32)
    t2h_ref[0:N, 0:7] = h2
    t2h_ref[0:N, 7:8] = a_s2
    t2d_ref[...] = jnp.zeros((NP, 16), _f32)
    t2d_ref[0:N, 0:1] = a_d2
    c2_ref[...] = jnp.full((1, 128), c2, _f32)


def _tc3_body(s2p_ref, t2h_ref, t2d_ref, c2_ref, b2_ref, out_ref):
    s = s2p_ref[0] + s2p_ref[1]
    h2 = t2h_ref[0:N, 0:7]
    a_s2 = t2h_ref[0:N, 7:8]
    a_d2 = t2d_ref[0:N, 0:1]
    c2 = c2_ref[0, 0]
    z = a_s2 + a_d2
    ex_self = jnp.exp(jnp.maximum(z, 0.2 * z) - c2)
    num = s[0:N, 0:7] + ex_self * h2
    den = s[0:N, 7:8] + ex_self
    o = num / den + b2_ref[...]
    m = jnp.max(o, axis=1, keepdims=True)
    zz = o - m
    out_ref[...] = zz - jnp.log(jnp.sum(jnp.exp(zz), axis=1, keepdims=True))


def _make_sc(width, edge_fn):
    @functools.partial(
        pl.kernel,
        mesh=_SC_MESH,
        compiler_params=_SC_PARAMS,
        out_type=jax.ShapeDtypeStruct((2, NP, width), _f32),
        scratch_types=[
            pltpu.VMEM((CHUNKS_PER_SUB, CHUNK), _i32),
            pltpu.VMEM((CHUNKS_PER_SUB, CHUNK), _i32),
            pltpu.VMEM((CHUNK, width), _f32),
            pltpu.VMEM((CHUNK, width), _f32),
            pltpu.VMEM((CHUNK, 16), _f32),
            pltpu.VMEM((CHUNK, 16), _f32),
            pltpu.VMEM((CHUNK,), _i32),
            pltpu.VMEM((CHUNK,), _i32),
            pltpu.VMEM((CHUNK,), _i32),
            pltpu.VMEM((CHUNK,), _i32),
            pltpu.VMEM((CHUNK, width), _f32),
            pltpu.VMEM((CHUNK, width), _f32),
            pltpu.VMEM((16,), _f32),
            pltpu.VMEM((16,), _f32),
            pltpu.VMEM_SHARED((NP, width), _f32),
            pltpu.SemaphoreType.DMA,
            pltpu.SemaphoreType.DMA,
            pltpu.SemaphoreType.DMA,
            pltpu.SemaphoreType.DMA,
        ],
    )
    def sck(th_hbm, td_hbm, src_hbm, dst_hbm, cvec_hbm, zeros_hbm,
            out_hbm, sstage, dstage, sbuf0, sbuf1, dbuf0, dbuf1, idxd0,
            idxd1, idxd2, idxd3, obuf0, obuf1, exb, cbuf, acc, gsem0,
            gsem1, ssem0, ssem1):
        cidx = lax.axis_index("c")
        sidx = lax.axis_index("s")
        wid = cidx * 16 + sidx

        sbufs = (sbuf0, sbuf1)
        dbufs = (dbuf0, dbuf1)
        idxds = (idxd0, idxd1, idxd2, idxd3)
        obufs = (obuf0, obuf1)
        gsems = (gsem0, gsem1)
        ssems = (ssem0, ssem1)

        r0 = sidx * ROWS_PER_TILE
        pltpu.sync_copy(zeros_hbm.at[pl.ds(r0, ROWS_PER_TILE)],
                        acc.at[pl.ds(r0, ROWS_PER_TILE)])
        pltpu.sync_copy(cvec_hbm, cbuf)
        pltpu.sync_copy(src_hbm.at[wid], sstage)
        pltpu.sync_copy(dst_hbm.at[wid], dstage)
        plsc.subcore_barrier()

        def start_gathers(c, p, d):
            pltpu.async_copy(th_hbm.at[sstage.at[c]], sbufs[p], gsems[p])
            pltpu.async_copy(td_hbm.at[dstage.at[c]], dbufs[p], gsems[p])
            pltpu.async_copy(dst_hbm.at[wid, c], idxds[d], gsems[p])

        def wait_gathers(c, p, d):
            pltpu.make_async_copy(th_hbm.at[sstage.at[c]], sbufs[p],
                                  gsems[p]).wait()
            pltpu.make_async_copy(td_hbm.at[dstage.at[c]], dbufs[p],
                                  gsems[p]).wait()
            pltpu.make_async_copy(dst_hbm.at[wid, c], idxds[d],
                                  gsems[p]).wait()

        def wait_scatter(p, d):
            pltpu.make_async_copy(obufs[p], acc.at[idxds[d]],
                                  ssems[p]).wait()

        def do_chunk(c, p, d, scatter_wait, prefetch):
            wait_gathers(c, p, d)

            @pl.when(scatter_wait)
            def _():
                wait_scatter(p, d)

            def edge_body(i, carry):
                for u in range(4):
                    edge_fn(4 * i + u, sbufs[p], dbufs[p], obufs[p], cbuf,
                            exb)
                return carry

            lax.fori_loop(0, CHUNK // 4, edge_body, 0)
            pltpu.async_copy(obufs[p], acc.at[idxds[d]], ssems[p], add=True)

            @pl.when(prefetch)
            def _():
                start_gathers(c + 2, p, (d + 2) % 4)

        start_gathers(0, 0, 0)
        start_gathers(1, 1, 1)

        def iter_body(g, carry):
            c0 = 4 * g
            last = g >= (CHUNKS_PER_SUB // 4) - 1
            do_chunk(c0 + 0, 0, 0, g > 0, True)
            do_chunk(c0 + 1, 1, 1, g > 0, True)
            do_chunk(c0 + 2, 0, 2, True, jnp.logical_not(last))
            do_chunk(c0 + 3, 1, 3, True, jnp.logical_not(last))
            return carry

        lax.fori_loop(0, CHUNKS_PER_SUB // 4, iter_body, 0)
        wait_scatter(0, 2)
        wait_scatter(1, 3)
        plsc.subcore_barrier()
        pltpu.sync_copy(acc.at[pl.ds(r0, ROWS_PER_TILE)],
                        out_hbm.at[cidx, pl.ds(r0, ROWS_PER_TILE)])

    return sck


_GDN = lax.GatherDimensionNumbers(offset_dims=(), collapsed_slice_dims=(0,),
                                  start_index_map=(0,))


def _vgather(vec, idx):
    return lax.gather(vec, idx[:, None], _GDN, (1,),
                      mode=lax.GatherScatterMode.PROMISE_IN_BOUNDS)


def _edge1(i, sbuf, dbuf, obuf, cbuf, exb):
    cv = cbuf[...]
    ge8 = lax.iota(_i32, 16) // 8
    sv4 = sbuf[i, pl.ds(64, 16)]
    dv = dbuf[i, pl.ds(0, 16)]
    z = sv4 + dv
    ex = jnp.exp(jnp.maximum(z, 0.2 * z) - cv)
    for k in range(4):
        exk = _vgather(ex, ge8 + 2 * k)
        obuf[i, pl.ds(16 * k, 16)] = sbuf[i, pl.ds(16 * k, 16)] * exk
    obuf[i, pl.ds(64, 16)] = ex


def _edge2(i, sbuf, dbuf, obuf, cbuf, exb):
    cv = cbuf[...]
    iot = lax.iota(_i32, 16)
    idx7 = jnp.full((16,), 7, _i32)
    idx0 = jnp.zeros((16,), _i32)
    mask7 = jnp.clip(7 - iot, 0, 1).astype(_f32)
    one7 = (jnp.clip(8 - iot, 0, 1)
            - jnp.clip(7 - iot, 0, 1)).astype(_f32)
    srow = sbuf[i, pl.ds(0, 16)]
    drow = dbuf[i, pl.ds(0, 16)]
    asv = _vgather(srow, idx7)
    adv = _vgather(drow, idx0)
    z = asv + adv
    ex = jnp.exp(jnp.maximum(z, 0.2 * z) - cv)
    obuf[i, pl.ds(0, 16)] = (srow * mask7 + one7) * ex


_SCK1 = _make_sc(80, _edge1)
_SCK2 = _make_sc(16, _edge2)


def kernel(x, edge_index, W1, att_src1, att_dst1, b1, W2, att_src2, att_dst2,
           b2):
    lane = jnp.arange(64)
    head = lane // 8
    blk = (head[:, None] == jnp.arange(8)[None, :]).astype(_f32)
    as_m = blk * att_src1.reshape(64)[:, None]
    ad_m = blk * att_dst1.reshape(64)[:, None]
    exp8 = blk.T
    as2_m = att_src2.reshape(7, 1)
    ad2_m = att_dst2.reshape(7, 1)

    pad = jnp.full((EP - E,), N, _i32)
    src3d = jnp.concatenate([edge_index[0].astype(_i32), pad]).reshape(
        NSUB, CHUNKS_PER_SUB, CHUNK)
    dst3d = jnp.concatenate([edge_index[1].astype(_i32), pad]).reshape(
        NSUB, CHUNKS_PER_SUB, CHUNK)

    zeros80 = jnp.zeros((NP, 80), _f32)
    zeros16 = jnp.zeros((NP, 16), _f32)

    t1h, t1d, c1 = pl.pallas_call(
        _tc1_body,
        out_shape=[
            jax.ShapeDtypeStruct((NP, 80), _f32),
            jax.ShapeDtypeStruct((NP, 16), _f32),
            jax.ShapeDtypeStruct((1, 128), _f32),
        ],
    )(x, W1, as_m, ad_m)
    c1v = c1[0, :16]

    s1p = _SCK1(t1h, t1d, src3d, dst3d, c1v, zeros80)

    t2h, t2d, c2 = pl.pallas_call(
        _tc2_body,
        out_shape=[
            jax.ShapeDtypeStruct((NP, 16), _f32),
            jax.ShapeDtypeStruct((NP, 16), _f32),
            jax.ShapeDtypeStruct((1, 128), _f32),
        ],
    )(s1p, t1h, t1d, c1, b1, W2, as2_m, ad2_m, exp8)
    c2v = c2[0, :16]

    s2p = _SCK2(t2h, t2d, src3d, dst3d, c2v, zeros16)

    out = pl.pallas_call(
        _tc3_body,
        out_shape=jax.ShapeDtypeStruct((N, 7), _f32),
    )(s2p, t2h, t2d, c2, b2)
    return out
